# baseline (speedup 1.0000x reference)
import functools

import jax
import jax.numpy as jnp
import numpy as np
from jax import lax
from jax.experimental import pallas as pl
from jax.experimental.pallas import tpu as pltpu
from jax.experimental.pallas import tpu_sc as plsc

_TARGET_ITEMS = (5, 17, 123, 999, 4242, 10000, 25000, 50000, 75000, 99999)
_K = 10
_ALPHA = 1.0
_ITEMS_LIMIT = 60
_NT = len(_TARGET_ITEMS)

_N_ROWS = 100000
_DIM = 64

_NC = 2
_NS = 16
_NW = _NC * _NS
_L = 16

_SC_START = 62112
_SC_ROWS_PER_TILE = 1184
_SC_CHUNK = 592
_SC_N_CHUNKS = _SC_ROWS_PER_TILE // _SC_CHUNK
_SC_GROUPS = _SC_CHUNK // _L
_SC_ROWS = _NW * _SC_ROWS_PER_TILE


def _sc_norms_body(emb_hbm, out_hbm, row_bufs, norm_bufs, in_sems, out_sems):
    cid = lax.axis_index("c")
    sid = lax.axis_index("s")
    wid = sid * _NC + cid
    base = _SC_START + wid * _SC_ROWS_PER_TILE
    lane = lax.iota(jnp.int32, _L)
    stride = lane * _DIM

    starts = [base + t * _SC_CHUNK for t in range(_SC_N_CHUNKS)]

    def in_copy(t):
        return pltpu.async_copy(
            emb_hbm.at[pl.ds(starts[t] * _DIM, _SC_CHUNK * _DIM)],
            row_bufs[t % 2],
            in_sems[t % 2],
        )

    out_cps = [None] * _SC_N_CHUNKS
    cur = in_copy(0)
    for t in range(_SC_N_CHUNKS):
        if t + 1 < _SC_N_CHUNKS:
            nxt = in_copy(t + 1)
        cur.wait()
        if t >= 2:
            out_cps[t - 2].wait()
        rows_v = row_bufs[t % 2]
        norms_v = norm_bufs[t % 2]

        def group_body(g, c2):
            gbase = g * (_L * _DIM) + stride
            accs = [jnp.zeros((_L,), jnp.float32) for _ in range(8)]
            for c in range(_DIM):
                v = plsc.load_gather(rows_v, [gbase + c])
                accs[c % 8] = accs[c % 8] + v * v
            acc = (
                (accs[0] + accs[1]) + (accs[2] + accs[3])
            ) + ((accs[4] + accs[5]) + (accs[6] + accs[7]))
            norms_v[pl.ds(g * _L, _L)] = acc
            return c2

        lax.fori_loop(0, _SC_GROUPS, group_body, 0)
        out_cps[t] = pltpu.async_copy(
            norms_v,
            out_hbm.at[pl.ds(starts[t] - _SC_START, _SC_CHUNK)],
            out_sems[t % 2],
        )
        if t + 1 < _SC_N_CHUNKS:
            cur = nxt
    for t in range(max(0, _SC_N_CHUNKS - 2), _SC_N_CHUNKS):
        out_cps[t].wait()


def _sc_norms(items_emb):
    mesh = plsc.VectorSubcoreMesh(core_axis_name="c", subcore_axis_name="s")
    f = functools.partial(
        pl.kernel,
        mesh=mesh,
        out_type=jax.ShapeDtypeStruct((_SC_ROWS,), jnp.float32),
        scratch_types=[
            [
                pltpu.VMEM((_SC_CHUNK * _DIM,), jnp.float32),
                pltpu.VMEM((_SC_CHUNK * _DIM,), jnp.float32),
            ],
            [
                pltpu.VMEM((_SC_CHUNK,), jnp.float32),
                pltpu.VMEM((_SC_CHUNK,), jnp.float32),
            ],
            [pltpu.SemaphoreType.DMA, pltpu.SemaphoreType.DMA],
            [pltpu.SemaphoreType.DMA, pltpu.SemaphoreType.DMA],
        ],
        compiler_params=pltpu.CompilerParams(needs_layout_passes=False),
    )(_sc_norms_body)
    return f(items_emb.reshape(_N_ROWS * _DIM))


_TC_ROWS_PAD = 62464
_TC_BLOCK = 15616
_TC_N_BLOCKS = _TC_ROWS_PAD // _TC_BLOCK


def _tc_norms_kernel(x_ref, o_ref):
    i = pl.program_id(0)
    x = x_ref[...]
    n2 = jnp.sum(x * x, axis=1)
    rid = i * _TC_BLOCK + lax.iota(jnp.int32, _TC_BLOCK)
    o_ref[0, 0, :] = jnp.where(rid < _SC_START, n2, -1.0)


def _tc_norms(items_emb):
    out = pl.pallas_call(
        _tc_norms_kernel,
        grid=(_TC_N_BLOCKS,),
        in_specs=[pl.BlockSpec((_TC_BLOCK, _DIM), lambda i: (i, 0))],
        out_specs=pl.BlockSpec((1, 1, _TC_BLOCK), lambda i: (i, 0, 0)),
        out_shape=jax.ShapeDtypeStruct(
            (_TC_N_BLOCKS, 1, _TC_BLOCK), jnp.float32
        ),
        compiler_params=pltpu.CompilerParams(
            dimension_semantics=("arbitrary",),
        ),
    )(items_emb)
    return out


_NRA = _TC_ROWS_PAD // 128
_NRB = _SC_ROWS // 128
_NR = _NRA + _NRB


def _finish_kernel(nA_ref, nB_ref, emb_any, out_ref, tgt_rows, row_a, row_b,
                   acc_s, tgt_sems, sem_a, sem_b):
    for j, t in enumerate(_TARGET_ITEMS):
        pltpu.make_async_copy(
            emb_any.at[pl.ds(t, 1), :],
            tgt_rows.at[pl.ds(j, 1), :],
            tgt_sems.at[j],
        ).start()

    r_iota = lax.broadcasted_iota(jnp.int32, (_NR, 128), 0)
    c_iota = lax.broadcasted_iota(jnp.int32, (_NR, 128), 1)
    fid = jnp.where(
        r_iota < _NRA,
        r_iota * 128 + c_iota,
        _SC_START + (r_iota - _NRA) * 128 + c_iota,
    )
    n2 = jnp.concatenate([nA_ref[...], nB_ref[...]], axis=0)
    valid = (r_iota >= _NRA) | (r_iota * 128 + c_iota < _SC_START)
    nm = jnp.where(valid, jnp.sqrt(n2), -1.0)

    out_ref[...] = jnp.zeros_like(out_ref)
    acc_s[...] = jnp.zeros_like(acc_s)

    bufs = (row_a, row_b)
    sems = (sem_a, sem_b)
    prev = None
    for k in range(_K):
        m = jnp.max(nm)
        idx = jnp.min(jnp.where(nm == m, fid, jnp.int32(2**31 - 1)))
        cp = pltpu.make_async_copy(
            emb_any.at[pl.ds(idx, 1), :], bufs[k % 2], sems[k % 2]
        )
        cp.start()
        if prev is not None:
            prev.wait()
            acc_s[...] += bufs[(k - 1) % 2][...]
        prev = cp
        nm = jnp.where(fid == idx, -jnp.inf, nm)
    prev.wait()
    acc_s[...] += bufs[(_K - 1) % 2][...]

    v = acc_s[...] / float(_K) * 10.0

    for j in range(_NT):
        pltpu.make_async_copy(
            emb_any.at[pl.ds(_TARGET_ITEMS[j], 1), :],
            tgt_rows.at[pl.ds(j, 1), :],
            tgt_sems.at[j],
        ).wait()
    nk = _ITEMS_LIMIT - 2 * _NT
    out_ref[pl.ds(nk, _NT), :] = (v - tgt_rows[...]) * _ALPHA


def _finish(norms2_tc, norms2_sc, items_emb):
    out_rows = 64
    return pl.pallas_call(
        _finish_kernel,
        in_specs=[
            pl.BlockSpec((_NRA, 128), lambda: (0, 0)),
            pl.BlockSpec((_NRB, 128), lambda: (0, 0)),
            pl.BlockSpec(memory_space=pl.ANY),
        ],
        out_specs=pl.BlockSpec((out_rows, _DIM), lambda: (0, 0)),
        out_shape=jax.ShapeDtypeStruct((out_rows, _DIM), jnp.float32),
        scratch_shapes=[
            pltpu.VMEM((_NT, _DIM), jnp.float32),
            pltpu.VMEM((1, _DIM), jnp.float32),
            pltpu.VMEM((1, _DIM), jnp.float32),
            pltpu.VMEM((1, _DIM), jnp.float32),
            pltpu.SemaphoreType.DMA((_NT,)),
            pltpu.SemaphoreType.DMA,
            pltpu.SemaphoreType.DMA,
        ],
    )(
        norms2_tc.reshape(_NRA, 128),
        norms2_sc.reshape(_NRB, 128),
        items_emb,
    )


@jax.jit
def kernel(items_emb):
    norms2_sc = _sc_norms(items_emb)
    norms2_tc = _tc_norms(items_emb)
    upd = _finish(norms2_tc, norms2_sc, items_emb)
    num_keep = _ITEMS_LIMIT - 2 * _NT
    kept = [i for i in range(_N_ROWS) if i not in _TARGET_ITEMS][:num_keep]
    chosen_items = jnp.asarray(list(kept) + list(_TARGET_ITEMS), dtype=jnp.int32)
    return chosen_items, upd[: num_keep + _NT]


# speedup vs baseline: 1.0013x; 1.0013x over previous
import functools

import jax
import jax.numpy as jnp
import numpy as np
from jax import lax
from jax.experimental import pallas as pl
from jax.experimental.pallas import tpu as pltpu
from jax.experimental.pallas import tpu_sc as plsc

_TARGET_ITEMS = (5, 17, 123, 999, 4242, 10000, 25000, 50000, 75000, 99999)
_K = 10
_ALPHA = 1.0
_ITEMS_LIMIT = 60
_NT = len(_TARGET_ITEMS)

_N_ROWS = 100000
_DIM = 64

_NC = 2
_NS = 16
_NW = _NC * _NS
_L = 16

_SC_START = 62112
_SC_ROWS_PER_TILE = 1184
_SC_CHUNK = 592
_SC_N_CHUNKS = _SC_ROWS_PER_TILE // _SC_CHUNK
_SC_GROUPS = _SC_CHUNK // _L
_SC_ROWS = _NW * _SC_ROWS_PER_TILE


def _sc_norms_body(emb_hbm, out_hbm, row_bufs, norm_bufs, in_sems, out_sems):
    cid = lax.axis_index("c")
    sid = lax.axis_index("s")
    wid = sid * _NC + cid
    base = _SC_START + wid * _SC_ROWS_PER_TILE
    lane = lax.iota(jnp.int32, _L)
    stride = lane * _DIM

    starts = [base + t * _SC_CHUNK for t in range(_SC_N_CHUNKS)]

    def in_copy(t):
        return pltpu.async_copy(
            emb_hbm.at[pl.ds(starts[t] * _DIM, _SC_CHUNK * _DIM)],
            row_bufs[t % 2],
            in_sems[t % 2],
        )

    out_cps = [None] * _SC_N_CHUNKS
    cur = in_copy(0)
    for t in range(_SC_N_CHUNKS):
        if t + 1 < _SC_N_CHUNKS:
            nxt = in_copy(t + 1)
        cur.wait()
        if t >= 2:
            out_cps[t - 2].wait()
        rows_v = row_bufs[t % 2]
        norms_v = norm_bufs[t % 2]

        def group_body(g, c2):
            gbase = g * (_L * _DIM) + stride
            accs = [jnp.zeros((_L,), jnp.float32) for _ in range(8)]
            for c in range(_DIM):
                v = plsc.load_gather(rows_v, [gbase + c])
                accs[c % 8] = accs[c % 8] + v * v
            acc = (
                (accs[0] + accs[1]) + (accs[2] + accs[3])
            ) + ((accs[4] + accs[5]) + (accs[6] + accs[7]))
            norms_v[pl.ds(g * _L, _L)] = acc
            return c2

        lax.fori_loop(0, _SC_GROUPS, group_body, 0)
        out_cps[t] = pltpu.async_copy(
            norms_v,
            out_hbm.at[pl.ds(starts[t] - _SC_START, _SC_CHUNK)],
            out_sems[t % 2],
        )
        if t + 1 < _SC_N_CHUNKS:
            cur = nxt
    for t in range(max(0, _SC_N_CHUNKS - 2), _SC_N_CHUNKS):
        out_cps[t].wait()


def _sc_norms(items_emb):
    mesh = plsc.VectorSubcoreMesh(core_axis_name="c", subcore_axis_name="s")
    f = functools.partial(
        pl.kernel,
        mesh=mesh,
        out_type=jax.ShapeDtypeStruct((_SC_ROWS,), jnp.float32),
        scratch_types=[
            [
                pltpu.VMEM((_SC_CHUNK * _DIM,), jnp.float32),
                pltpu.VMEM((_SC_CHUNK * _DIM,), jnp.float32),
            ],
            [
                pltpu.VMEM((_SC_CHUNK,), jnp.float32),
                pltpu.VMEM((_SC_CHUNK,), jnp.float32),
            ],
            [pltpu.SemaphoreType.DMA, pltpu.SemaphoreType.DMA],
            [pltpu.SemaphoreType.DMA, pltpu.SemaphoreType.DMA],
        ],
        compiler_params=pltpu.CompilerParams(needs_layout_passes=False),
    )(_sc_norms_body)
    return f(items_emb.reshape(_N_ROWS * _DIM))


_TC_ROWS_PAD = 62464
_TC_BLOCK = 15616
_TC_N_BLOCKS = _TC_ROWS_PAD // _TC_BLOCK


def _tc_norms_kernel(x_ref, o_ref):
    i = pl.program_id(0)
    x = x_ref[...]
    n2 = jnp.sum(x * x, axis=1)
    rid = i * _TC_BLOCK + lax.iota(jnp.int32, _TC_BLOCK)
    o_ref[0, 0, :] = jnp.where(rid < _SC_START, n2, -1.0)


def _tc_norms(items_emb):
    out = pl.pallas_call(
        _tc_norms_kernel,
        grid=(_TC_N_BLOCKS,),
        in_specs=[pl.BlockSpec((_TC_BLOCK, _DIM), lambda i: (i, 0))],
        out_specs=pl.BlockSpec((1, 1, _TC_BLOCK), lambda i: (i, 0, 0)),
        out_shape=jax.ShapeDtypeStruct(
            (_TC_N_BLOCKS, 1, _TC_BLOCK), jnp.float32
        ),
        compiler_params=pltpu.CompilerParams(
            dimension_semantics=("arbitrary",),
        ),
    )(items_emb)
    return out


_NRA = _TC_ROWS_PAD // 128
_NRB = _SC_ROWS // 128
_NR = _NRA + _NRB


def _finish_kernel(nA_ref, nB_ref, emb_any, out_ref, tgt_rows, row_a, row_b,
                   acc_s, tgt_sems, sem_a, sem_b):
    for j, t in enumerate(_TARGET_ITEMS):
        pltpu.make_async_copy(
            emb_any.at[pl.ds(t, 1), :],
            tgt_rows.at[pl.ds(j, 1), :],
            tgt_sems.at[j],
        ).start()

    r_iota = lax.broadcasted_iota(jnp.int32, (_NR, 128), 0)
    c_iota = lax.broadcasted_iota(jnp.int32, (_NR, 128), 1)
    fid = jnp.where(
        r_iota < _NRA,
        r_iota * 128 + c_iota,
        _SC_START + (r_iota - _NRA) * 128 + c_iota,
    )
    n2 = jnp.concatenate([nA_ref[...], nB_ref[...]], axis=0)
    valid = (r_iota >= _NRA) | (r_iota * 128 + c_iota < _SC_START)
    nm = jnp.where(valid, jnp.sqrt(n2), -1.0)

    out_ref[...] = jnp.zeros_like(out_ref)
    acc_s[...] = jnp.zeros_like(acc_s)

    bufs = (row_a, row_b)
    sems = (sem_a, sem_b)
    prev = None
    for k in range(_K):
        m = jnp.max(nm)
        idx = jnp.min(jnp.where(nm == m, fid, jnp.int32(2**31 - 1)))
        cp = pltpu.make_async_copy(
            emb_any.at[pl.ds(idx, 1), :], bufs[k % 2], sems[k % 2]
        )
        cp.start()
        if prev is not None:
            prev.wait()
            acc_s[...] += bufs[(k - 1) % 2][...]
        prev = cp
        nm = jnp.where(fid == idx, -jnp.inf, nm)
    prev.wait()
    acc_s[...] += bufs[(_K - 1) % 2][...]

    v = acc_s[...] / float(_K) * 10.0

    for j in range(_NT):
        pltpu.make_async_copy(
            emb_any.at[pl.ds(_TARGET_ITEMS[j], 1), :],
            tgt_rows.at[pl.ds(j, 1), :],
            tgt_sems.at[j],
        ).wait()
    nk = _ITEMS_LIMIT - 2 * _NT
    out_ref[pl.ds(nk, _NT), :] = (v - tgt_rows[...]) * _ALPHA


def _finish(norms2_tc, norms2_sc, items_emb):
    out_rows = 64
    return pl.pallas_call(
        _finish_kernel,
        in_specs=[
            pl.BlockSpec((_NRA, 128), lambda: (0, 0)),
            pl.BlockSpec((_NRB, 128), lambda: (0, 0)),
            pl.BlockSpec(memory_space=pl.ANY),
        ],
        out_specs=pl.BlockSpec((out_rows, _DIM), lambda: (0, 0)),
        out_shape=jax.ShapeDtypeStruct((out_rows, _DIM), jnp.float32),
        scratch_shapes=[
            pltpu.VMEM((_NT, _DIM), jnp.float32),
            pltpu.VMEM((1, _DIM), jnp.float32),
            pltpu.VMEM((1, _DIM), jnp.float32),
            pltpu.VMEM((1, _DIM), jnp.float32),
            pltpu.SemaphoreType.DMA((_NT,)),
            pltpu.SemaphoreType.DMA,
            pltpu.SemaphoreType.DMA,
        ],
    )(
        norms2_tc.reshape(_NRA, 128),
        norms2_sc.reshape(_NRB, 128),
        items_emb,
    )


@jax.jit
def kernel(items_emb):
    norms2_tc = _tc_norms(items_emb)
    norms2_sc = _sc_norms(items_emb)
    upd = _finish(norms2_tc, norms2_sc, items_emb)
    num_keep = _ITEMS_LIMIT - 2 * _NT
    kept = [i for i in range(_N_ROWS) if i not in _TARGET_ITEMS][:num_keep]
    chosen_items = jnp.asarray(list(kept) + list(_TARGET_ITEMS), dtype=jnp.int32)
    return chosen_items, upd[: num_keep + _NT]


# speedup vs baseline: 2.0673x; 2.0646x over previous
import functools

import jax
import jax.numpy as jnp
import numpy as np
from jax.experimental import pallas as pl
from jax.experimental.pallas import tpu as pltpu

_TARGET_ITEMS = (5, 17, 123, 999, 4242, 10000, 25000, 50000, 75000, 99999)
_K = 10
_ALPHA = 1.0
_ITEMS_LIMIT = 60
_NT = len(_TARGET_ITEMS)

_N_ROWS = 100000
_DIM = 64
_BLOCK_ROWS = 20000
_N_BLOCKS = (_N_ROWS + _BLOCK_ROWS - 1) // _BLOCK_ROWS


def _attack_kernel(
    x_ref, emb_any, out_ref, norms_s, tgt_rows, row_a, row_b, acc_s,
    tgt_sems, sem_a, sem_b
):
    i = pl.program_id(0)

    @pl.when(i == 0)
    def _():
        for j, t in enumerate(_TARGET_ITEMS):
            pltpu.make_async_copy(
                emb_any.at[pl.ds(t, 1), :],
                tgt_rows.at[pl.ds(j, 1), :],
                tgt_sems.at[j],
            ).start()

    x = x_ref[...]
    n2 = jnp.sum(x * x, axis=1)
    rid = i * _BLOCK_ROWS + jax.lax.iota(jnp.int32, _BLOCK_ROWS)
    norms_s[i, :] = jnp.where(rid < _N_ROWS, jnp.sqrt(n2), -1.0)

    @pl.when(i == _N_BLOCKS - 1)
    def _():
        out_ref[...] = jnp.zeros_like(out_ref)
        acc_s[...] = jnp.zeros_like(acc_s)

        fid = (
            jax.lax.broadcasted_iota(jnp.int32, (_N_BLOCKS, _BLOCK_ROWS), 0)
            * _BLOCK_ROWS
            + jax.lax.broadcasted_iota(jnp.int32, (_N_BLOCKS, _BLOCK_ROWS), 1)
        )
        nm = norms_s[...]

        bufs = (row_a, row_b)
        sems = (sem_a, sem_b)
        prev = None
        for k in range(_K):
            m = jnp.max(nm)
            idx = jnp.min(jnp.where(nm == m, fid, jnp.int32(2**31 - 1)))
            cp = pltpu.make_async_copy(
                emb_any.at[pl.ds(idx, 1), :], bufs[k % 2], sems[k % 2]
            )
            cp.start()
            if prev is not None:
                prev.wait()
                acc_s[...] += bufs[(k - 1) % 2][...]
            prev = cp
            nm = jnp.where(fid == idx, -jnp.inf, nm)
        prev.wait()
        acc_s[...] += bufs[(_K - 1) % 2][...]

        v = acc_s[...] / float(_K) * 10.0

        for j in range(_NT):
            pltpu.make_async_copy(
                emb_any.at[pl.ds(_TARGET_ITEMS[j], 1), :],
                tgt_rows.at[pl.ds(j, 1), :],
                tgt_sems.at[j],
            ).wait()
        nk = _ITEMS_LIMIT - 2 * _NT
        out_ref[pl.ds(nk, _NT), :] = (v - tgt_rows[...]) * _ALPHA


@jax.jit
def kernel(items_emb):
    out_rows = 64
    upd = pl.pallas_call(
        _attack_kernel,
        grid=(_N_BLOCKS,),
        in_specs=[
            pl.BlockSpec((_BLOCK_ROWS, _DIM), lambda i: (i, 0)),
            pl.BlockSpec(memory_space=pl.ANY),
        ],
        out_specs=pl.BlockSpec((out_rows, _DIM), lambda i: (0, 0)),
        out_shape=jax.ShapeDtypeStruct((out_rows, _DIM), jnp.float32),
        scratch_shapes=[
            pltpu.VMEM((_N_BLOCKS, _BLOCK_ROWS), jnp.float32),
            pltpu.VMEM((_NT, _DIM), jnp.float32),
            pltpu.VMEM((1, _DIM), jnp.float32),
            pltpu.VMEM((1, _DIM), jnp.float32),
            pltpu.VMEM((1, _DIM), jnp.float32),
            pltpu.SemaphoreType.DMA((_NT,)),
            pltpu.SemaphoreType.DMA,
            pltpu.SemaphoreType.DMA,
        ],
        compiler_params=pltpu.CompilerParams(
            dimension_semantics=("arbitrary",),
        ),
    )(items_emb, items_emb)

    num_keep = _ITEMS_LIMIT - 2 * _NT
    kept = [i for i in range(_N_ROWS) if i not in _TARGET_ITEMS][:num_keep]
    chosen_items = jnp.asarray(list(kept) + list(_TARGET_ITEMS), dtype=jnp.int32)
    return chosen_items, upd[: num_keep + _NT]
